# baseline (speedup 1.0000x reference)
import dataclasses
import functools

import jax
import jax.numpy as jnp
from jax import lax
from jax.experimental import pallas as pl
from jax.experimental.pallas import tpu as pltpu
from jax.experimental.pallas import tpu_sc as plsc

N = 10000
E = 320000
D = 128
G = 512
D_OUT = 16

NC = 2
NS = 16
NW = NC * NS
EW = E // NW
CH = 80
NCH = EW // CH
NBUF = 4
ECH = 64
EPW = 10240
ENCH = EPW // ECH
SB = 32
NSB = ENCH // SB
NP = 10240
RW = NP // NS

def _deg_body(dst_hbm, zeros_hbm, out_hbm, dst_v, deg_v):
    c = lax.axis_index("c")
    s = lax.axis_index("s")
    wid = c * NS + s
    pltpu.sync_copy(dst_hbm.at[wid], dst_v)
    pltpu.sync_copy(zeros_hbm, deg_v)
    ones = jnp.full((16,), 1.0, jnp.float32)

    @pl.loop(0, NCH)
    def _(i):
        for j in range(CH // 16):
            idx = dst_v[i, pl.ds(j * 16, 16)]
            plsc.addupdate_scatter(deg_v, [idx], ones)

    pltpu.sync_copy(deg_v, out_hbm.at[pl.ds(wid * N, N)])


def _sc_compiler_params():
    cp = pltpu.CompilerParams()
    if "needs_layout_passes" in pltpu.CompilerParams.__dataclass_fields__:
        cp = dataclasses.replace(cp, needs_layout_passes=False)
    return cp


@functools.cache
def _deg_kernel():
    mesh = plsc.VectorSubcoreMesh(core_axis_name="c", subcore_axis_name="s",
                                  num_cores=NC, num_subcores=NS)
    return pl.kernel(
        _deg_body,
        out_type=jax.ShapeDtypeStruct((NW * N,), jnp.float32),
        mesh=mesh,
        compiler_params=_sc_compiler_params(),
        scratch_types=[
            pltpu.VMEM((NCH, CH), jnp.int32),
            pltpu.VMEM((N,), jnp.float32),
        ],
    )


def _edge_body(h_hbm, src_hbm, dst_hbm, zeros_hbm,
               out0_hbm, out1_hbm, src_v, dst_v, bufs, acc, sems):
    c = lax.axis_index("c")
    s = lax.axis_index("s")
    wid = c * NS + s
    pltpu.sync_copy(zeros_hbm, acc.at[pl.ds(s * RW, RW)])
    plsc.subcore_barrier()

    def _gather(chunk, b):
        return pltpu.async_copy(
            h_hbm.at[src_v.at[chunk]], bufs[b], sems[b])

    def _gather_wait(chunk, b):
        pltpu.make_async_copy(
            h_hbm.at[src_v.at[chunk]], bufs[b], sems[b]).wait()

    def _scatter(p, b):
        return pltpu.async_copy(
            bufs[b], acc.at[dst_v.at[p]], sems[NBUF + b], add=True)

    def _scatter_wait(p, b):
        pltpu.make_async_copy(
            bufs[b], acc.at[dst_v.at[p]], sems[NBUF + b]).wait()

    @pl.loop(0, NSB)
    def _(o):
        pltpu.sync_copy(src_hbm.at[wid, o], src_v)
        pltpu.sync_copy(dst_hbm.at[wid, o], dst_v)
        for b in range(NBUF):
            _gather(b, b)

        @pl.loop(0, SB - NBUF, step=NBUF)
        def _(p):
            for b in range(NBUF):
                _gather_wait(p + b, b)
                _scatter(p + b, b)
            for b in range(NBUF):
                _scatter_wait(p + b, b)
                _gather(p + NBUF + b, b)

        for b in range(NBUF):
            _gather_wait(SB - NBUF + b, b)
            _scatter(SB - NBUF + b, b)
        for b in range(NBUF):
            _scatter_wait(SB - NBUF + b, b)

    plsc.subcore_barrier()
    rows = pl.ds(s * RW, RW)

    @pl.when(c == 0)
    def _():
        pltpu.sync_copy(acc.at[rows], out0_hbm.at[rows])

    @pl.when(c == 1)
    def _():
        pltpu.sync_copy(acc.at[rows], out1_hbm.at[rows])


@functools.cache
def _edge_kernel():
    mesh = plsc.VectorSubcoreMesh(core_axis_name="c", subcore_axis_name="s",
                                  num_cores=NC, num_subcores=NS)
    return pl.kernel(
        _edge_body,
        out_type=[
            jax.ShapeDtypeStruct((NP, D), jnp.float32),
            jax.ShapeDtypeStruct((NP, D), jnp.float32),
        ],
        mesh=mesh,
        scratch_types=[
            pltpu.VMEM((SB, ECH), jnp.int32),
            pltpu.VMEM((SB, ECH), jnp.int32),
            [pltpu.VMEM((ECH, D), jnp.float32) for _ in range(NBUF)],
            pltpu.VMEM_SHARED((NP, D), jnp.float32),
            [pltpu.SemaphoreType.DMA for _ in range(2 * NBUF)],
        ],
    )


def _mm_scale_body(x_ref, w_ref, degt_ref, out_ref, dinv_ref):
    d = jnp.sum(degt_ref[...], axis=1, keepdims=True) + 1.0
    dinv = jnp.broadcast_to(lax.rsqrt(d), out_ref.shape)
    dinv_ref[...] = dinv
    h = jnp.dot(x_ref[...], w_ref[...], preferred_element_type=jnp.float32)
    out_ref[...] = h * dinv


def _mm_scale_tc(x, w, deg_t):
    blk = 1000
    return pl.pallas_call(
        _mm_scale_body,
        grid=(N // blk,),
        in_specs=[
            pl.BlockSpec((blk, D), lambda i: (i, 0)),
            pl.BlockSpec((D, D), lambda i: (0, 0)),
            pl.BlockSpec((blk, NW), lambda i: (i, 0)),
        ],
        out_specs=[
            pl.BlockSpec((blk, D), lambda i: (i, 0)),
            pl.BlockSpec((blk, D), lambda i: (i, 0)),
        ],
        out_shape=[
            jax.ShapeDtypeStruct((N, D), jnp.float32),
            jax.ShapeDtypeStruct((N, D), jnp.float32),
        ],
    )(x, w, deg_t)


def _combine_body(sa_ref, sb_ref, hp_ref, dinv_ref, b_ref, w_ref, out_ref):
    pre = (sa_ref[...] + sb_ref[...] + hp_ref[...]) * dinv_ref[...] + b_ref[...]
    h = jnp.maximum(pre, 0.0)
    out_ref[...] = (
        jnp.dot(h, w_ref[...], preferred_element_type=jnp.float32)
        * dinv_ref[...])


def _combine_tc(sa, sb, hp, dinv_b, b1, w2):
    blk = 1000
    return pl.pallas_call(
        _combine_body,
        grid=(N // blk,),
        in_specs=[
            pl.BlockSpec((blk, D), lambda i: (i, 0)),
            pl.BlockSpec((blk, D), lambda i: (i, 0)),
            pl.BlockSpec((blk, D), lambda i: (i, 0)),
            pl.BlockSpec((blk, D), lambda i: (i, 0)),
            pl.BlockSpec((1, D), lambda i: (0, 0)),
            pl.BlockSpec((D, D), lambda i: (0, 0)),
        ],
        out_specs=pl.BlockSpec((blk, D), lambda i: (i, 0)),
        out_shape=jax.ShapeDtypeStruct((N, D), jnp.float32),
    )(sa, sb, hp, dinv_b, b1, w2)


_PBLK = 1000


def _pool_body(sa_ref, sb_ref, hp_ref, dinv_ref, b_ref, batch_ref,
               w1_ref, b1_ref, w2_ref, b2_ref, out_ref, sums_ref, cnt_ref):
    pre = (sa_ref[...] + sb_ref[...] + hp_ref[...]) * dinv_ref[...] + b_ref[...]
    h2 = jnp.maximum(pre, 0.0)
    bids = batch_ref[0]
    gids = lax.broadcasted_iota(jnp.int32, (G, _PBLK), 0)
    onehot_t = (gids == bids).astype(jnp.float32)
    psums = jnp.dot(onehot_t, h2, preferred_element_type=jnp.float32)
    pcnt = jnp.broadcast_to(
        jnp.sum(onehot_t, axis=1, keepdims=True), (G, D))

    @pl.when(pl.program_id(0) == 0)
    def _():
        sums_ref[...] = psums
        cnt_ref[...] = pcnt

    @pl.when(pl.program_id(0) != 0)
    def _():
        sums_ref[...] += psums
        cnt_ref[...] += pcnt

    @pl.when(pl.program_id(0) == N // _PBLK - 1)
    def _():
        g = sums_ref[...] / jnp.maximum(cnt_ref[...], 1.0)
        a = jnp.maximum(
            jnp.dot(g, w1_ref[...], preferred_element_type=jnp.float32)
            + b1_ref[...], 0.0)
        z = (jnp.dot(a, w2_ref[...], preferred_element_type=jnp.float32)
             + b2_ref[...])
        m = jnp.max(z, axis=1, keepdims=True)
        ez = jnp.exp(z - m)
        out_ref[...] = (z - m) - jnp.log(jnp.sum(ez, axis=1, keepdims=True))


def _pool_tc(sa, sb, hp, dinv_b, b2, batch3, fc1_w, fc1_b, fc2_w, fc2_b):
    return pl.pallas_call(
        _pool_body,
        grid=(N // _PBLK,),
        in_specs=[
            pl.BlockSpec((_PBLK, D), lambda i: (i, 0)),
            pl.BlockSpec((_PBLK, D), lambda i: (i, 0)),
            pl.BlockSpec((_PBLK, D), lambda i: (i, 0)),
            pl.BlockSpec((_PBLK, D), lambda i: (i, 0)),
            pl.BlockSpec((1, D), lambda i: (0, 0)),
            pl.BlockSpec((1, 1, _PBLK), lambda i: (i, 0, 0)),
            pl.BlockSpec((D, D), lambda i: (0, 0)),
            pl.BlockSpec((1, D), lambda i: (0, 0)),
            pl.BlockSpec((D, D_OUT), lambda i: (0, 0)),
            pl.BlockSpec((1, D_OUT), lambda i: (0, 0)),
        ],
        out_specs=pl.BlockSpec((G, D_OUT), lambda i: (0, 0)),
        out_shape=jax.ShapeDtypeStruct((G, D_OUT), jnp.float32),
        scratch_shapes=[
            pltpu.VMEM((G, D), jnp.float32),
            pltpu.VMEM((G, D), jnp.float32),
        ],
    )(sa, sb, hp, dinv_b, b2, batch3, fc1_w, fc1_b, fc2_w, fc2_b)


def kernel(x, edge_index, batch, W1, b1, W2, b2, fc1_W, fc1_b, fc2_W, fc2_b):
    src_i = edge_index[0].astype(jnp.int32)
    dst_i = edge_index[1].astype(jnp.int32)
    pad = EPW - EW
    srcE = jnp.concatenate(
        [src_i.reshape(NW, EW),
         jnp.zeros((NW, pad), jnp.int32)], axis=1).reshape(NW, NSB, SB, ECH)
    dstE = jnp.concatenate(
        [dst_i.reshape(NW, EW),
         jnp.full((NW, pad), NP - 1, jnp.int32)], axis=1
    ).reshape(NW, NSB, SB, ECH)
    dst3 = dst_i.reshape(NW, NCH, CH)
    batch3 = batch.astype(jnp.int32).reshape(N // _PBLK, 1, _PBLK)
    zrows = jnp.zeros((RW, D), jnp.float32)
    zdeg = jnp.zeros((N,), jnp.float32)

    deg_parts = _deg_kernel()(dst3, zdeg)
    deg_t = deg_parts.reshape(NW, N).T

    h1p, dinv_b = _mm_scale_tc(x, W1, deg_t)
    s1a, s1b = _edge_kernel()(h1p, srcE, dstE, zrows)
    h2p = _combine_tc(s1a, s1b, h1p, dinv_b, b1.reshape(1, D), W2)
    s2a, s2b = _edge_kernel()(h2p, srcE, dstE, zrows)
    return _pool_tc(s2a, s2b, h2p, dinv_b, b2.reshape(1, D), batch3,
                    fc1_W, fc1_b.reshape(1, D),
                    fc2_W, fc2_b.reshape(1, D_OUT))


# speedup vs baseline: 1.0004x; 1.0004x over previous
import dataclasses
import functools

import jax
import jax.numpy as jnp
from jax import lax
from jax.experimental import pallas as pl
from jax.experimental.pallas import tpu as pltpu
from jax.experimental.pallas import tpu_sc as plsc

N = 10000
E = 320000
D = 128
G = 512
D_OUT = 16

NC = 2
NS = 16
NW = NC * NS
EW = E // NW
CH = 80
NCH = EW // CH
NBUF = 4
ECH = 64
EPW = 10240
ENCH = EPW // ECH
SB = 32
NSB = ENCH // SB
NP = 10240
RW = NP // NS

def _deg_body(dst_hbm, zeros_hbm, out_hbm, dst_v, deg_v):
    c = lax.axis_index("c")
    s = lax.axis_index("s")
    wid = c * NS + s
    pltpu.sync_copy(dst_hbm.at[wid], dst_v)
    pltpu.sync_copy(zeros_hbm, deg_v)
    ones = jnp.full((16,), 1.0, jnp.float32)

    @pl.loop(0, NCH)
    def _(i):
        for j in range(CH // 16):
            idx = dst_v[i, pl.ds(j * 16, 16)]
            plsc.addupdate_scatter(deg_v, [idx], ones)

    pltpu.sync_copy(deg_v, out_hbm.at[pl.ds(wid * N, N)])


def _sc_compiler_params():
    cp = pltpu.CompilerParams()
    if "needs_layout_passes" in pltpu.CompilerParams.__dataclass_fields__:
        cp = dataclasses.replace(cp, needs_layout_passes=False)
    return cp


@functools.cache
def _deg_kernel():
    mesh = plsc.VectorSubcoreMesh(core_axis_name="c", subcore_axis_name="s",
                                  num_cores=NC, num_subcores=NS)
    return pl.kernel(
        _deg_body,
        out_type=jax.ShapeDtypeStruct((NW * N,), jnp.float32),
        mesh=mesh,
        compiler_params=_sc_compiler_params(),
        scratch_types=[
            pltpu.VMEM((NCH, CH), jnp.int32),
            pltpu.VMEM((N,), jnp.float32),
        ],
    )


def _edge_body(h_hbm, src_hbm, dst_hbm, zeros_hbm,
               out0_hbm, out1_hbm, src_v, dst_v, bufs, acc, sems):
    c = lax.axis_index("c")
    s = lax.axis_index("s")
    wid = c * NS + s
    pltpu.sync_copy(zeros_hbm, acc.at[pl.ds(s * RW, RW)])
    plsc.subcore_barrier()

    def _gather(chunk, b):
        return pltpu.async_copy(
            h_hbm.at[src_v.at[chunk]], bufs[b], sems[b])

    def _gather_wait(chunk, b):
        pltpu.make_async_copy(
            h_hbm.at[src_v.at[chunk]], bufs[b], sems[b]).wait()

    def _scatter(p, b):
        return pltpu.async_copy(
            bufs[b], acc.at[dst_v.at[p]], sems[NBUF + b], add=True)

    def _scatter_wait(p, b):
        pltpu.make_async_copy(
            bufs[b], acc.at[dst_v.at[p]], sems[NBUF + b]).wait()

    @pl.loop(0, NSB)
    def _(o):
        pltpu.sync_copy(src_hbm.at[wid, o], src_v)
        pltpu.sync_copy(dst_hbm.at[wid, o], dst_v)
        for b in range(NBUF):
            _gather(b, b)

        @pl.loop(0, SB - NBUF, step=NBUF)
        def _(p):
            for b in range(NBUF):
                _gather_wait(p + b, b)
                _scatter(p + b, b)
            for b in range(NBUF):
                _scatter_wait(p + b, b)
                _gather(p + NBUF + b, b)

        for b in range(NBUF):
            _gather_wait(SB - NBUF + b, b)
            _scatter(SB - NBUF + b, b)
        for b in range(NBUF):
            _scatter_wait(SB - NBUF + b, b)

    plsc.subcore_barrier()
    rows = pl.ds(s * RW, RW)

    @pl.when(c == 0)
    def _():
        pltpu.sync_copy(acc.at[rows], out0_hbm.at[rows])

    @pl.when(c == 1)
    def _():
        pltpu.sync_copy(acc.at[rows], out1_hbm.at[rows])


@functools.cache
def _edge_kernel():
    mesh = plsc.VectorSubcoreMesh(core_axis_name="c", subcore_axis_name="s",
                                  num_cores=NC, num_subcores=NS)
    return pl.kernel(
        _edge_body,
        out_type=[
            jax.ShapeDtypeStruct((NP, D), jnp.float32),
            jax.ShapeDtypeStruct((NP, D), jnp.float32),
        ],
        mesh=mesh,
        scratch_types=[
            pltpu.VMEM((SB, ECH), jnp.int32),
            pltpu.VMEM((SB, ECH), jnp.int32),
            [pltpu.VMEM((ECH, D), jnp.float32) for _ in range(NBUF)],
            pltpu.VMEM_SHARED((NP, D), jnp.float32),
            [pltpu.SemaphoreType.DMA for _ in range(2 * NBUF)],
        ],
    )


def _mm_scale_body(x_ref, w_ref, degt_ref, out_ref, dinv_ref):
    d = jnp.sum(degt_ref[...], axis=1, keepdims=True) + 1.0
    dinv = jnp.broadcast_to(lax.rsqrt(d), out_ref.shape)
    dinv_ref[...] = dinv
    h = jnp.dot(x_ref[...], w_ref[...], preferred_element_type=jnp.float32)
    out_ref[...] = h * dinv


def _mm_scale_tc(x, w, deg_t):
    blk = 1000
    return pl.pallas_call(
        _mm_scale_body,
        grid=(N // blk,),
        in_specs=[
            pl.BlockSpec((blk, D), lambda i: (i, 0)),
            pl.BlockSpec((D, D), lambda i: (0, 0)),
            pl.BlockSpec((blk, NW), lambda i: (i, 0)),
        ],
        out_specs=[
            pl.BlockSpec((blk, D), lambda i: (i, 0)),
            pl.BlockSpec((blk, D), lambda i: (i, 0)),
        ],
        out_shape=[
            jax.ShapeDtypeStruct((N, D), jnp.float32),
            jax.ShapeDtypeStruct((N, D), jnp.float32),
        ],
    )(x, w, deg_t)


def _combine_body(sa_ref, sb_ref, hp_ref, dinv_ref, b_ref, w_ref, out_ref):
    pre = (sa_ref[...] + sb_ref[...] + hp_ref[...]) * dinv_ref[...] + b_ref[...]
    h = jnp.maximum(pre, 0.0)
    out_ref[...] = (
        jnp.dot(h, w_ref[...], preferred_element_type=jnp.float32)
        * dinv_ref[...])


def _combine_tc(sa, sb, hp, dinv_b, b1, w2):
    blk = 1000
    return pl.pallas_call(
        _combine_body,
        grid=(N // blk,),
        in_specs=[
            pl.BlockSpec((blk, D), lambda i: (i, 0)),
            pl.BlockSpec((blk, D), lambda i: (i, 0)),
            pl.BlockSpec((blk, D), lambda i: (i, 0)),
            pl.BlockSpec((blk, D), lambda i: (i, 0)),
            pl.BlockSpec((1, D), lambda i: (0, 0)),
            pl.BlockSpec((D, D), lambda i: (0, 0)),
        ],
        out_specs=pl.BlockSpec((blk, D), lambda i: (i, 0)),
        out_shape=jax.ShapeDtypeStruct((N, D), jnp.float32),
    )(sa, sb, hp, dinv_b, b1, w2)


_PBLK = 1000


def _pool_body(sa_ref, sb_ref, hp_ref, dinv_ref, b_ref, batch_ref,
               w1_ref, b1_ref, w2_ref, b2_ref, out_ref, sums_ref, cnt_ref):
    pre = (sa_ref[...] + sb_ref[...] + hp_ref[...]) * dinv_ref[...] + b_ref[...]
    h2 = jnp.maximum(pre, 0.0)
    bids = batch_ref[0]
    gids = lax.broadcasted_iota(jnp.int32, (G, _PBLK), 0)
    onehot_t = (gids == bids).astype(jnp.float32)
    psums = jnp.dot(onehot_t, h2, preferred_element_type=jnp.float32)
    pcnt = jnp.broadcast_to(
        jnp.sum(onehot_t, axis=1, keepdims=True), (G, D))

    @pl.when(pl.program_id(0) == 0)
    def _():
        sums_ref[...] = psums
        cnt_ref[...] = pcnt

    @pl.when(pl.program_id(0) != 0)
    def _():
        sums_ref[...] += psums
        cnt_ref[...] += pcnt

    @pl.when(pl.program_id(0) == N // _PBLK - 1)
    def _():
        g = sums_ref[...] / jnp.maximum(cnt_ref[...], 1.0)
        a = jnp.maximum(
            jnp.dot(g, w1_ref[...], preferred_element_type=jnp.float32)
            + b1_ref[...], 0.0)
        z = (jnp.dot(a, w2_ref[...], preferred_element_type=jnp.float32)
             + b2_ref[...])
        m = jnp.max(z, axis=1, keepdims=True)
        ez = jnp.exp(z - m)
        out_ref[...] = (z - m) - jnp.log(jnp.sum(ez, axis=1, keepdims=True))


def _pool_tc(sa, sb, hp, dinv_b, b2, batch3, fc1_w, fc1_b, fc2_w, fc2_b):
    return pl.pallas_call(
        _pool_body,
        grid=(N // _PBLK,),
        in_specs=[
            pl.BlockSpec((_PBLK, D), lambda i: (i, 0)),
            pl.BlockSpec((_PBLK, D), lambda i: (i, 0)),
            pl.BlockSpec((_PBLK, D), lambda i: (i, 0)),
            pl.BlockSpec((_PBLK, D), lambda i: (i, 0)),
            pl.BlockSpec((1, D), lambda i: (0, 0)),
            pl.BlockSpec((1, 1, _PBLK), lambda i: (i, 0, 0)),
            pl.BlockSpec((D, D), lambda i: (0, 0)),
            pl.BlockSpec((1, D), lambda i: (0, 0)),
            pl.BlockSpec((D, D_OUT), lambda i: (0, 0)),
            pl.BlockSpec((1, D_OUT), lambda i: (0, 0)),
        ],
        out_specs=pl.BlockSpec((G, D_OUT), lambda i: (0, 0)),
        out_shape=jax.ShapeDtypeStruct((G, D_OUT), jnp.float32),
        scratch_shapes=[
            pltpu.VMEM((G, D), jnp.float32),
            pltpu.VMEM((G, D), jnp.float32),
        ],
    )(sa, sb, hp, dinv_b, b2, batch3, fc1_w, fc1_b, fc2_w, fc2_b)


def kernel(x, edge_index, batch, W1, b1, W2, b2, fc1_W, fc1_b, fc2_W, fc2_b):
    src_i = edge_index[0].astype(jnp.int32)
    dst_i = edge_index[1].astype(jnp.int32)
    pad = EPW - EW
    srcE = jnp.concatenate(
        [src_i.reshape(NW, EW),
         jnp.zeros((NW, pad), jnp.int32)], axis=1).reshape(NW, NSB, SB, ECH)
    dstE = jnp.concatenate(
        [dst_i.reshape(NW, EW),
         jnp.broadcast_to(N + jnp.arange(pad, dtype=jnp.int32),
                          (NW, pad))], axis=1
    ).reshape(NW, NSB, SB, ECH)
    dst3 = dst_i.reshape(NW, NCH, CH)
    batch3 = batch.astype(jnp.int32).reshape(N // _PBLK, 1, _PBLK)
    zrows = jnp.zeros((RW, D), jnp.float32)
    zdeg = jnp.zeros((N,), jnp.float32)

    deg_parts = _deg_kernel()(dst3, zdeg)
    deg_t = deg_parts.reshape(NW, N).T

    h1p, dinv_b = _mm_scale_tc(x, W1, deg_t)
    s1a, s1b = _edge_kernel()(h1p, srcE, dstE, zrows)
    h2p = _combine_tc(s1a, s1b, h1p, dinv_b, b1.reshape(1, D), W2)
    s2a, s2b = _edge_kernel()(h2p, srcE, dstE, zrows)
    return _pool_tc(s2a, s2b, h2p, dinv_b, b2.reshape(1, D), batch3,
                    fc1_W, fc1_b.reshape(1, D),
                    fc2_W, fc2_b.reshape(1, D_OUT))


# speedup vs baseline: 2.6662x; 2.6652x over previous
import dataclasses
import functools

import jax
import jax.numpy as jnp
from jax import lax
from jax.experimental import pallas as pl
from jax.experimental.pallas import tpu as pltpu
from jax.experimental.pallas import tpu_sc as plsc

N = 10000
E = 320000
D = 128
G = 512
D_OUT = 16

NC = 2
NS = 16
NW = NC * NS
EW = E // NW
CH = 80
NCH = EW // CH
NBUF = 5
ECH = 40
ENCH = EW // ECH
SB = 50
NSB = ENCH // SB
NP = 10240
RW = NP // NS

def _deg_body(dst_hbm, zeros_hbm, out_hbm, dst_v, deg_v):
    c = lax.axis_index("c")
    s = lax.axis_index("s")
    wid = c * NS + s
    pltpu.sync_copy(dst_hbm.at[wid], dst_v)
    pltpu.sync_copy(zeros_hbm, deg_v)
    ones = jnp.full((16,), 1.0, jnp.float32)

    @pl.loop(0, NCH)
    def _(i):
        for j in range(CH // 16):
            idx = dst_v[i, pl.ds(j * 16, 16)]
            plsc.addupdate_scatter(deg_v, [idx], ones)

    pltpu.sync_copy(deg_v, out_hbm.at[pl.ds(wid * N, N)])


def _sc_compiler_params():
    cp = pltpu.CompilerParams()
    if "needs_layout_passes" in pltpu.CompilerParams.__dataclass_fields__:
        cp = dataclasses.replace(cp, needs_layout_passes=False)
    return cp


@functools.cache
def _deg_kernel():
    mesh = plsc.VectorSubcoreMesh(core_axis_name="c", subcore_axis_name="s",
                                  num_cores=NC, num_subcores=NS)
    return pl.kernel(
        _deg_body,
        out_type=jax.ShapeDtypeStruct((NW * N,), jnp.float32),
        mesh=mesh,
        compiler_params=_sc_compiler_params(),
        scratch_types=[
            pltpu.VMEM((NCH, CH), jnp.int32),
            pltpu.VMEM((N,), jnp.float32),
        ],
    )


def _edge_body(h_hbm, src_hbm, dst_hbm, zeros_hbm,
               out0_hbm, out1_hbm, src_v, dst_v, bufs, acc, sems):
    c = lax.axis_index("c")
    s = lax.axis_index("s")
    wid = c * NS + s
    pltpu.sync_copy(zeros_hbm, acc.at[pl.ds(s * RW, RW)])
    plsc.subcore_barrier()

    def _gather(chunk, b):
        return pltpu.async_copy(
            h_hbm.at[src_v.at[chunk]], bufs[b], sems[b])

    def _gather_wait(chunk, b):
        pltpu.make_async_copy(
            h_hbm.at[src_v.at[chunk]], bufs[b], sems[b]).wait()

    def _scatter(p, b):
        return pltpu.async_copy(
            bufs[b], acc.at[dst_v.at[p]], sems[NBUF + b], add=True)

    def _scatter_wait(p, b):
        pltpu.make_async_copy(
            bufs[b], acc.at[dst_v.at[p]], sems[NBUF + b]).wait()

    @pl.loop(0, NSB)
    def _(o):
        pltpu.sync_copy(src_hbm.at[wid, o], src_v)
        pltpu.sync_copy(dst_hbm.at[wid, o], dst_v)
        for b in range(NBUF):
            _gather(b, b)

        @pl.loop(0, SB - NBUF, step=NBUF)
        def _(p):
            for b in range(NBUF):
                _gather_wait(p + b, b)
                _scatter(p + b, b)
            for b in range(NBUF):
                _scatter_wait(p + b, b)
                _gather(p + NBUF + b, b)

        for b in range(NBUF):
            _gather_wait(SB - NBUF + b, b)
            _scatter(SB - NBUF + b, b)
        for b in range(NBUF):
            _scatter_wait(SB - NBUF + b, b)

    plsc.subcore_barrier()
    rows = pl.ds(s * RW, RW)

    @pl.when(c == 0)
    def _():
        pltpu.sync_copy(acc.at[rows], out0_hbm.at[rows])

    @pl.when(c == 1)
    def _():
        pltpu.sync_copy(acc.at[rows], out1_hbm.at[rows])


@functools.cache
def _edge_kernel():
    mesh = plsc.VectorSubcoreMesh(core_axis_name="c", subcore_axis_name="s",
                                  num_cores=NC, num_subcores=NS)
    return pl.kernel(
        _edge_body,
        out_type=[
            jax.ShapeDtypeStruct((NP, D), jnp.float32),
            jax.ShapeDtypeStruct((NP, D), jnp.float32),
        ],
        mesh=mesh,
        scratch_types=[
            pltpu.VMEM((SB, ECH), jnp.int32),
            pltpu.VMEM((SB, ECH), jnp.int32),
            [pltpu.VMEM((ECH, D), jnp.float32) for _ in range(NBUF)],
            pltpu.VMEM_SHARED((NP, D), jnp.float32),
            [pltpu.SemaphoreType.DMA for _ in range(2 * NBUF)],
        ],
    )


def _mm_scale_body(x_ref, w_ref, degt_ref, out_ref, dinv_ref):
    d = jnp.sum(degt_ref[...], axis=1, keepdims=True) + 1.0
    dinv = jnp.broadcast_to(lax.rsqrt(d), out_ref.shape)
    dinv_ref[...] = dinv
    h = jnp.dot(x_ref[...], w_ref[...], preferred_element_type=jnp.float32)
    out_ref[...] = h * dinv


def _mm_scale_tc(x, w, deg_t):
    blk = 1000
    return pl.pallas_call(
        _mm_scale_body,
        grid=(N // blk,),
        in_specs=[
            pl.BlockSpec((blk, D), lambda i: (i, 0)),
            pl.BlockSpec((D, D), lambda i: (0, 0)),
            pl.BlockSpec((blk, NW), lambda i: (i, 0)),
        ],
        out_specs=[
            pl.BlockSpec((blk, D), lambda i: (i, 0)),
            pl.BlockSpec((blk, D), lambda i: (i, 0)),
        ],
        out_shape=[
            jax.ShapeDtypeStruct((N, D), jnp.float32),
            jax.ShapeDtypeStruct((N, D), jnp.float32),
        ],
    )(x, w, deg_t)


def _combine_body(sa_ref, sb_ref, hp_ref, dinv_ref, b_ref, w_ref, out_ref):
    pre = (sa_ref[...] + sb_ref[...] + hp_ref[...]) * dinv_ref[...] + b_ref[...]
    h = jnp.maximum(pre, 0.0)
    out_ref[...] = (
        jnp.dot(h, w_ref[...], preferred_element_type=jnp.float32)
        * dinv_ref[...])


def _combine_tc(sa, sb, hp, dinv_b, b1, w2):
    blk = 1000
    return pl.pallas_call(
        _combine_body,
        grid=(N // blk,),
        in_specs=[
            pl.BlockSpec((blk, D), lambda i: (i, 0)),
            pl.BlockSpec((blk, D), lambda i: (i, 0)),
            pl.BlockSpec((blk, D), lambda i: (i, 0)),
            pl.BlockSpec((blk, D), lambda i: (i, 0)),
            pl.BlockSpec((1, D), lambda i: (0, 0)),
            pl.BlockSpec((D, D), lambda i: (0, 0)),
        ],
        out_specs=pl.BlockSpec((blk, D), lambda i: (i, 0)),
        out_shape=jax.ShapeDtypeStruct((N, D), jnp.float32),
    )(sa, sb, hp, dinv_b, b1, w2)


_PBLK = 1000


def _pool_body(sa_ref, sb_ref, hp_ref, dinv_ref, b_ref, batch_ref,
               w1_ref, b1_ref, w2_ref, b2_ref, out_ref, sums_ref, cnt_ref):
    pre = (sa_ref[...] + sb_ref[...] + hp_ref[...]) * dinv_ref[...] + b_ref[...]
    h2 = jnp.maximum(pre, 0.0)
    bids = batch_ref[0]
    gids = lax.broadcasted_iota(jnp.int32, (G, _PBLK), 0)
    onehot_t = (gids == bids).astype(jnp.float32)
    psums = jnp.dot(onehot_t, h2, preferred_element_type=jnp.float32)
    pcnt = jnp.broadcast_to(
        jnp.sum(onehot_t, axis=1, keepdims=True), (G, D))

    @pl.when(pl.program_id(0) == 0)
    def _():
        sums_ref[...] = psums
        cnt_ref[...] = pcnt

    @pl.when(pl.program_id(0) != 0)
    def _():
        sums_ref[...] += psums
        cnt_ref[...] += pcnt

    @pl.when(pl.program_id(0) == N // _PBLK - 1)
    def _():
        g = sums_ref[...] / jnp.maximum(cnt_ref[...], 1.0)
        a = jnp.maximum(
            jnp.dot(g, w1_ref[...], preferred_element_type=jnp.float32)
            + b1_ref[...], 0.0)
        z = (jnp.dot(a, w2_ref[...], preferred_element_type=jnp.float32)
             + b2_ref[...])
        m = jnp.max(z, axis=1, keepdims=True)
        ez = jnp.exp(z - m)
        out_ref[...] = (z - m) - jnp.log(jnp.sum(ez, axis=1, keepdims=True))


def _pool_tc(sa, sb, hp, dinv_b, b2, batch3, fc1_w, fc1_b, fc2_w, fc2_b):
    return pl.pallas_call(
        _pool_body,
        grid=(N // _PBLK,),
        in_specs=[
            pl.BlockSpec((_PBLK, D), lambda i: (i, 0)),
            pl.BlockSpec((_PBLK, D), lambda i: (i, 0)),
            pl.BlockSpec((_PBLK, D), lambda i: (i, 0)),
            pl.BlockSpec((_PBLK, D), lambda i: (i, 0)),
            pl.BlockSpec((1, D), lambda i: (0, 0)),
            pl.BlockSpec((1, 1, _PBLK), lambda i: (i, 0, 0)),
            pl.BlockSpec((D, D), lambda i: (0, 0)),
            pl.BlockSpec((1, D), lambda i: (0, 0)),
            pl.BlockSpec((D, D_OUT), lambda i: (0, 0)),
            pl.BlockSpec((1, D_OUT), lambda i: (0, 0)),
        ],
        out_specs=pl.BlockSpec((G, D_OUT), lambda i: (0, 0)),
        out_shape=jax.ShapeDtypeStruct((G, D_OUT), jnp.float32),
        scratch_shapes=[
            pltpu.VMEM((G, D), jnp.float32),
            pltpu.VMEM((G, D), jnp.float32),
        ],
    )(sa, sb, hp, dinv_b, b2, batch3, fc1_w, fc1_b, fc2_w, fc2_b)


def kernel(x, edge_index, batch, W1, b1, W2, b2, fc1_W, fc1_b, fc2_W, fc2_b):
    srcE = edge_index[0].astype(jnp.int32).reshape(NW, NSB, SB, ECH)
    dstE = edge_index[1].astype(jnp.int32).reshape(NW, NSB, SB, ECH)
    dst3 = edge_index[1].astype(jnp.int32).reshape(NW, NCH, CH)
    batch3 = batch.astype(jnp.int32).reshape(N // _PBLK, 1, _PBLK)
    zrows = jnp.zeros((RW, D), jnp.float32)
    zdeg = jnp.zeros((N,), jnp.float32)

    deg_parts = _deg_kernel()(dst3, zdeg)
    deg_t = deg_parts.reshape(NW, N).T

    h1p, dinv_b = _mm_scale_tc(x, W1, deg_t)
    s1a, s1b = _edge_kernel()(h1p, srcE, dstE, zrows)
    h2p = _combine_tc(s1a, s1b, h1p, dinv_b, b1.reshape(1, D), W2)
    s2a, s2b = _edge_kernel()(h2p, srcE, dstE, zrows)
    return _pool_tc(s2a, s2b, h2p, dinv_b, b2.reshape(1, D), batch3,
                    fc1_W, fc1_b.reshape(1, D),
                    fc2_W, fc2_b.reshape(1, D_OUT))
